# baseline (speedup 1.0000x reference)
import functools
import math

import jax
import jax.numpy as jnp
from jax.experimental import pallas as pl
from jax.experimental.pallas import tpu as pltpu

BB = 128
HB = 64


def _lstm3_kernel(x_ref,
                  wih0_ref, whh0_ref, b0_ref,
                  wih1_ref, whh1_ref, b1_ref,
                  wih2_ref, whh2_ref, b2_ref,
                  wlin_ref, blin_ref,
                  out_ref,
                  proj_scr, seq_scr,
                  *, T, H, unroll):
    def run_layer(whh_ref, store_seq):
        whh = whh_ref[...]

        def step(t, carry):
            ha, ca, hb, cb = carry
            ra = pl.multiple_of(t * BB, BB)
            rb = pl.multiple_of(t * BB + HB, HB)
            ga = proj_scr[pl.ds(ra, HB), :] + jnp.dot(
                ha, whh, preferred_element_type=jnp.float32)
            gb = proj_scr[pl.ds(rb, HB), :] + jnp.dot(
                hb, whh, preferred_element_type=jnp.float32)

            def update(g, c):
                sig = 0.5 * jnp.tanh(g[:, :3 * H] * 0.5) + 0.5
                gg = jnp.tanh(g[:, 3 * H:])
                i_g = sig[:, 0 * H:1 * H]
                f_g = sig[:, 1 * H:2 * H]
                o_g = sig[:, 2 * H:3 * H]
                c_new = f_g * c + i_g * gg
                h_new = o_g * jnp.tanh(c_new)
                return h_new, c_new

            ha, ca = update(ga, ca)
            hb, cb = update(gb, cb)
            if store_seq:
                seq_scr[pl.ds(ra, HB), :] = ha
                seq_scr[pl.ds(rb, HB), :] = hb
            return ha, ca, hb, cb

        zero = jnp.zeros((HB, H), jnp.float32)
        return jax.lax.fori_loop(0, T, step, (zero, zero, zero, zero),
                                 unroll=unroll)

    proj_scr[...] = jnp.dot(x_ref[...], wih0_ref[...],
                            preferred_element_type=jnp.float32) + b0_ref[...]
    run_layer(whh0_ref, store_seq=True)

    proj_scr[...] = jnp.dot(seq_scr[...], wih1_ref[...],
                            preferred_element_type=jnp.float32) + b1_ref[...]
    run_layer(whh1_ref, store_seq=True)

    proj_scr[...] = jnp.dot(seq_scr[...], wih2_ref[...],
                            preferred_element_type=jnp.float32) + b2_ref[...]
    ha, _, hb, _ = run_layer(whh2_ref, store_seq=False)

    wlin = wlin_ref[...]
    out_ref[pl.ds(0, HB), :] = (
        jnp.sum(ha * wlin, axis=-1, keepdims=True) + blin_ref[...])
    out_ref[pl.ds(HB, HB), :] = (
        jnp.sum(hb * wlin, axis=-1, keepdims=True) + blin_ref[...])


def _forward(x_blocks, lstm_params, lin_w_row, lin_b, *, T, B_pad, H, I):
    nb = B_pad // BB
    unroll = math.gcd(T, 8)
    body = functools.partial(_lstm3_kernel, T=T, H=H, unroll=unroll)

    def full2d(shape):
        return pl.BlockSpec(shape, lambda i: (0,) * len(shape))

    in_specs = [pl.BlockSpec((None, T * BB, I), lambda i: (i, 0, 0))]
    args = [x_blocks]
    for (w_ih, w_hh, bias) in lstm_params:
        in_specs += [full2d(w_ih.shape), full2d(w_hh.shape),
                     full2d(bias.shape)]
        args += [w_ih, w_hh, bias]
    in_specs += [full2d((1, H)), full2d((1, 1))]
    args += [lin_w_row, lin_b]

    return pl.pallas_call(
        body,
        out_shape=jax.ShapeDtypeStruct((B_pad, 1), jnp.float32),
        grid=(nb,),
        in_specs=in_specs,
        out_specs=pl.BlockSpec((BB, 1), lambda i: (i, 0)),
        scratch_shapes=[
            pltpu.VMEM((T * BB, 4 * H), jnp.float32),
            pltpu.VMEM((T * BB, H), jnp.float32),
        ],
        compiler_params=pltpu.CompilerParams(
            dimension_semantics=("parallel",),
            vmem_limit_bytes=56 * 1024 * 1024),
    )(*args)


@jax.jit
def kernel(x, lstm0_w_ih_t, lstm0_w_hh_t, lstm0_bias,
           lstm1_w_ih_t, lstm1_w_hh_t, lstm1_bias,
           lstm2_w_ih_t, lstm2_w_hh_t, lstm2_bias,
           lin_w_row, lin_b):
    B, T, I = x.shape
    H = lstm0_w_hh_t.shape[0]

    perm = jnp.concatenate([jnp.arange(2 * H), jnp.arange(3 * H, 4 * H),
                            jnp.arange(2 * H, 3 * H)])
    lstm_params = [
        (lstm0_w_ih_t[:, perm], lstm0_w_hh_t[:, perm], lstm0_bias[:, perm]),
        (lstm1_w_ih_t[:, perm], lstm1_w_hh_t[:, perm], lstm1_bias[:, perm]),
        (lstm2_w_ih_t[:, perm], lstm2_w_hh_t[:, perm], lstm2_bias[:, perm]),
    ]

    B_pad = ((B + BB - 1) // BB) * BB
    nb = B_pad // BB
    x_tm = jnp.transpose(x.astype(jnp.float32), (1, 0, 2))
    x_tm = jnp.pad(x_tm, ((0, 0), (0, B_pad - B), (0, 0)))
    x_blocks = (x_tm.reshape(T, nb, BB, I)
                .transpose(1, 0, 2, 3)
                .reshape(nb, T * BB, I))

    out = _forward(x_blocks, lstm_params, lin_w_row, lin_b,
                   T=T, B_pad=B_pad, H=H, I=I)
    return out[:B, 0]


# speedup vs baseline: 1.7979x; 1.7979x over previous
import functools
import math

import jax
import jax.numpy as jnp
import numpy as np
from jax.experimental import pallas as pl
from jax.experimental.pallas import tpu as pltpu
from jax.sharding import Mesh, PartitionSpec as P

try:
    from jax.experimental.shard_map import shard_map as _shard_map
except ImportError:
    _shard_map = jax.shard_map

BB = 128
HB = 64


def _lstm3_kernel(x_ref,
                  wih0_ref, whh0_ref, b0_ref,
                  wih1_ref, whh1_ref, b1_ref,
                  wih2_ref, whh2_ref, b2_ref,
                  wlin_ref, blin_ref,
                  out_ref,
                  proj_scr, seq_scr,
                  *, T, H, unroll):
    def run_layer(whh_ref, store_seq):
        whh = whh_ref[...]

        def step(t, carry):
            ha, ca, hb, cb = carry
            ra = pl.multiple_of(t * BB, BB)
            rb = pl.multiple_of(t * BB + HB, HB)
            ga = proj_scr[pl.ds(ra, HB), :] + jnp.dot(
                ha, whh, preferred_element_type=jnp.float32)
            gb = proj_scr[pl.ds(rb, HB), :] + jnp.dot(
                hb, whh, preferred_element_type=jnp.float32)

            def update(g, c):
                sig = 0.5 * jnp.tanh(g[:, :3 * H] * 0.5) + 0.5
                gg = jnp.tanh(g[:, 3 * H:])
                i_g = sig[:, 0 * H:1 * H]
                f_g = sig[:, 1 * H:2 * H]
                o_g = sig[:, 2 * H:3 * H]
                c_new = f_g * c + i_g * gg
                h_new = o_g * jnp.tanh(c_new)
                return h_new, c_new

            ha, ca = update(ga, ca)
            hb, cb = update(gb, cb)
            if store_seq:
                seq_scr[pl.ds(ra, HB), :] = ha
                seq_scr[pl.ds(rb, HB), :] = hb
            return ha, ca, hb, cb

        zero = jnp.zeros((HB, H), jnp.float32)
        return jax.lax.fori_loop(0, T, step, (zero, zero, zero, zero),
                                 unroll=unroll)

    proj_scr[...] = jnp.dot(x_ref[...], wih0_ref[...],
                            preferred_element_type=jnp.float32) + b0_ref[...]
    run_layer(whh0_ref, store_seq=True)

    proj_scr[...] = jnp.dot(seq_scr[...], wih1_ref[...],
                            preferred_element_type=jnp.float32) + b1_ref[...]
    run_layer(whh1_ref, store_seq=True)

    proj_scr[...] = jnp.dot(seq_scr[...], wih2_ref[...],
                            preferred_element_type=jnp.float32) + b2_ref[...]
    ha, _, hb, _ = run_layer(whh2_ref, store_seq=False)

    wlin = wlin_ref[...]
    out_ref[pl.ds(0, HB), :] = (
        jnp.sum(ha * wlin, axis=-1, keepdims=True) + blin_ref[...])
    out_ref[pl.ds(HB, HB), :] = (
        jnp.sum(hb * wlin, axis=-1, keepdims=True) + blin_ref[...])


def _forward(x_blocks, lstm_params, lin_w_row, lin_b, *, T, B_pad, H, I):
    nb = B_pad // BB
    unroll = math.gcd(T, 8)
    body = functools.partial(_lstm3_kernel, T=T, H=H, unroll=unroll)

    def full2d(shape):
        return pl.BlockSpec(shape, lambda i: (0,) * len(shape))

    in_specs = [pl.BlockSpec((None, T * BB, I), lambda i: (i, 0, 0))]
    args = [x_blocks]
    for (w_ih, w_hh, bias) in lstm_params:
        in_specs += [full2d(w_ih.shape), full2d(w_hh.shape),
                     full2d(bias.shape)]
        args += [w_ih, w_hh, bias]
    in_specs += [full2d((1, H)), full2d((1, 1))]
    args += [lin_w_row, lin_b]

    return pl.pallas_call(
        body,
        out_shape=jax.ShapeDtypeStruct((B_pad, 1), jnp.float32),
        grid=(nb,),
        in_specs=in_specs,
        out_specs=pl.BlockSpec((BB, 1), lambda i: (i, 0)),
        scratch_shapes=[
            pltpu.VMEM((T * BB, 4 * H), jnp.float32),
            pltpu.VMEM((T * BB, H), jnp.float32),
        ],
        compiler_params=pltpu.CompilerParams(
            dimension_semantics=("parallel",),
            vmem_limit_bytes=56 * 1024 * 1024),
    )(*args)


@jax.jit
def kernel(x, lstm0_w_ih_t, lstm0_w_hh_t, lstm0_bias,
           lstm1_w_ih_t, lstm1_w_hh_t, lstm1_bias,
           lstm2_w_ih_t, lstm2_w_hh_t, lstm2_bias,
           lin_w_row, lin_b):
    B, T, I = x.shape
    H = lstm0_w_hh_t.shape[0]

    perm = jnp.concatenate([jnp.arange(2 * H), jnp.arange(3 * H, 4 * H),
                            jnp.arange(2 * H, 3 * H)])
    lstm_params = [
        (lstm0_w_ih_t[:, perm], lstm0_w_hh_t[:, perm], lstm0_bias[:, perm]),
        (lstm1_w_ih_t[:, perm], lstm1_w_hh_t[:, perm], lstm1_bias[:, perm]),
        (lstm2_w_ih_t[:, perm], lstm2_w_hh_t[:, perm], lstm2_bias[:, perm]),
    ]

    B_pad = ((B + BB - 1) // BB) * BB
    nb = B_pad // BB
    x_tm = jnp.transpose(x.astype(jnp.float32), (1, 0, 2))
    x_tm = jnp.pad(x_tm, ((0, 0), (0, B_pad - B), (0, 0)))
    x_blocks = (x_tm.reshape(T, nb, BB, I)
                .transpose(1, 0, 2, 3)
                .reshape(nb, T * BB, I))

    flat_w = [w for lp in lstm_params for w in lp] + [lin_w_row, lin_b]

    def fwd(xb, *ws):
        lps = [tuple(ws[3 * l:3 * l + 3]) for l in range(3)]
        return _forward(xb, lps, ws[9], ws[10],
                        T=T, B_pad=xb.shape[0] * BB, H=H, I=I)

    devs = jax.devices()
    if len(devs) >= 2 and nb % 2 == 0:
        mesh = Mesh(np.array(devs[:2]), ("d",))
        try:
            fwd = _shard_map(
                fwd, mesh=mesh,
                in_specs=(P("d"),) + (P(),) * len(flat_w),
                out_specs=P("d"), check_vma=False)
        except TypeError:
            fwd = _shard_map(
                fwd, mesh=mesh,
                in_specs=(P("d"),) + (P(),) * len(flat_w),
                out_specs=P("d"), check_rep=False)
    out = fwd(x_blocks, *flat_w)
    return out[:B, 0]
